# Initial kernel scaffold; baseline (speedup 1.0000x reference)
import functools

import jax
import jax.numpy as jnp
from jax import lax
from jax.experimental import pallas as pl
from jax.experimental.pallas import tpu as pltpu
from jax.experimental.pallas import tpu_sc as plsc

N = 10000
E = 320000
D = 128

NC = 2
NS = 16
NW = NC * NS
CHUNK = 100
CPW = E // (NW * CHUNK)
ROWS_PT = N // NS

_SC_MESH = plsc.VectorSubcoreMesh(core_axis_name="c", subcore_axis_name="s")


def _deg_body(dst_hbm, zeros_hbm, ones_hbm, out_hbm, idx_v, ones_v, deg_sh,
              sem):
    c = lax.axis_index("c")
    s = lax.axis_index("s")
    wid = c * NS + s
    pltpu.sync_copy(ones_hbm, ones_v)
    pltpu.sync_copy(zeros_hbm.at[pl.ds(s * ROWS_PT, ROWS_PT)],
                    deg_sh.at[pl.ds(s * ROWS_PT, ROWS_PT)])
    cp = pltpu.async_copy(dst_hbm.at[pl.ds(wid * CPW, CPW)], idx_v, sem)
    cp.wait()
    plsc.subcore_barrier()

    def body(j, carry):
        pltpu.sync_copy(ones_v, deg_sh.at[idx_v.at[j]], add=True)
        return carry

    lax.fori_loop(0, CPW, body, 0)
    plsc.subcore_barrier()
    pltpu.sync_copy(deg_sh.at[pl.ds(s * ROWS_PT, ROWS_PT)],
                    out_hbm.at[c, pl.ds(s * ROWS_PT, ROWS_PT)])


_deg_kernel = pl.kernel(
    _deg_body,
    out_type=jax.ShapeDtypeStruct((NC, N, 1), jnp.float32),
    mesh=_SC_MESH,
    scratch_types=[
        pltpu.VMEM((CPW, CHUNK), jnp.int32),
        pltpu.VMEM((CHUNK, 1), jnp.float32),
        pltpu.VMEM_SHARED((N, 1), jnp.float32),
        pltpu.SemaphoreType.DMA,
    ],
)


def _agg_body(h_hbm, src_hbm, dst_hbm, zeros_hbm, out_hbm,
              isrc_v, idst_v, buf0, buf1, acc_sh, sem0, sem1, isem):
    c = lax.axis_index("c")
    s = lax.axis_index("s")
    wid = c * NS + s
    cp_s = pltpu.async_copy(src_hbm.at[pl.ds(wid * CPW, CPW)], isrc_v, isem)
    pltpu.sync_copy(zeros_hbm.at[pl.ds(s * ROWS_PT, ROWS_PT)],
                    acc_sh.at[pl.ds(s * ROWS_PT, ROWS_PT)])
    pltpu.sync_copy(dst_hbm.at[pl.ds(wid * CPW, CPW)], idst_v)
    cp_s.wait()
    plsc.subcore_barrier()

    def body(g, carry):
        j0 = 2 * g
        j1 = j0 + 1
        cp0 = pltpu.async_copy(h_hbm.at[isrc_v.at[j0]], buf0, sem0)
        cp1 = pltpu.async_copy(h_hbm.at[isrc_v.at[j1]], buf1, sem1)
        cp0.wait()
        pltpu.sync_copy(buf0, acc_sh.at[idst_v.at[j0]], add=True)
        cp1.wait()
        pltpu.sync_copy(buf1, acc_sh.at[idst_v.at[j1]], add=True)
        return carry

    lax.fori_loop(0, CPW // 2, body, 0)
    plsc.subcore_barrier()
    pltpu.sync_copy(acc_sh.at[pl.ds(s * ROWS_PT, ROWS_PT)],
                    out_hbm.at[c, pl.ds(s * ROWS_PT, ROWS_PT)])


_agg_kernel = pl.kernel(
    _agg_body,
    out_type=jax.ShapeDtypeStruct((NC, N, D), jnp.float32),
    mesh=_SC_MESH,
    scratch_types=[
        pltpu.VMEM((CPW, CHUNK), jnp.int32),
        pltpu.VMEM((CPW, CHUNK), jnp.int32),
        pltpu.VMEM((CHUNK, D), jnp.float32),
        pltpu.VMEM((CHUNK, D), jnp.float32),
        pltpu.VMEM_SHARED((N, D), jnp.float32),
        pltpu.SemaphoreType.DMA,
        pltpu.SemaphoreType.DMA,
        pltpu.SemaphoreType.DMA,
    ],
)


RB = 400
GRID = N // RB


def _tc1_body(x_ref, w_ref, dp_ref, hp_ref, dis_ref):
    dis = lax.rsqrt(dp_ref[0] + dp_ref[1] + 1.0)
    h = jnp.dot(x_ref[...], w_ref[...], preferred_element_type=jnp.float32)
    hp_ref[...] = h * dis
    dis_ref[...] = dis


_tc1 = pl.pallas_call(
    _tc1_body,
    grid=(GRID,),
    in_specs=[
        pl.BlockSpec((RB, D), lambda i: (i, 0)),
        pl.BlockSpec((D, D), lambda i: (0, 0)),
        pl.BlockSpec((NC, RB, 1), lambda i: (0, i, 0)),
    ],
    out_specs=[
        pl.BlockSpec((RB, D), lambda i: (i, 0)),
        pl.BlockSpec((RB, 1), lambda i: (i, 0)),
    ],
    out_shape=[
        jax.ShapeDtypeStruct((N, D), jnp.float32),
        jax.ShapeDtypeStruct((N, 1), jnp.float32),
    ],
)


def _tc2_body(p_ref, hp1_ref, dis_ref, b1_ref, w2_ref, hp2_ref):
    ssum = p_ref[0] + p_ref[1] + hp1_ref[...]
    h1 = jnp.maximum(ssum * dis_ref[...] + b1_ref[...], 0.0)
    h2 = jnp.dot(h1, w2_ref[...], preferred_element_type=jnp.float32)
    hp2_ref[...] = h2 * dis_ref[...]


_tc2 = pl.pallas_call(
    _tc2_body,
    grid=(GRID,),
    in_specs=[
        pl.BlockSpec((NC, RB, D), lambda i: (0, i, 0)),
        pl.BlockSpec((RB, D), lambda i: (i, 0)),
        pl.BlockSpec((RB, 1), lambda i: (i, 0)),
        pl.BlockSpec((1, D), lambda i: (0, 0)),
        pl.BlockSpec((D, D), lambda i: (0, 0)),
    ],
    out_specs=pl.BlockSpec((RB, D), lambda i: (i, 0)),
    out_shape=jax.ShapeDtypeStruct((N, D), jnp.float32),
)


def _tc3_body(p_ref, hp2_ref, dis_ref, b2_ref, out_ref):
    ssum = p_ref[0] + p_ref[1] + hp2_ref[...]
    out_ref[...] = ssum * dis_ref[...] + b2_ref[...]


_tc3 = pl.pallas_call(
    _tc3_body,
    grid=(GRID,),
    in_specs=[
        pl.BlockSpec((NC, RB, D), lambda i: (0, i, 0)),
        pl.BlockSpec((RB, D), lambda i: (i, 0)),
        pl.BlockSpec((RB, 1), lambda i: (i, 0)),
        pl.BlockSpec((1, D), lambda i: (0, 0)),
    ],
    out_specs=pl.BlockSpec((RB, D), lambda i: (i, 0)),
    out_shape=jax.ShapeDtypeStruct((N, D), jnp.float32),
)


@jax.jit
def kernel(x, edge_index, W1, b1, W2, b2):
    src2 = edge_index[0].reshape(E // CHUNK, CHUNK)
    dst2 = edge_index[1].reshape(E // CHUNK, CHUNK)
    zeros1 = jnp.zeros((N, 1), jnp.float32)
    zerosD = jnp.zeros((N, D), jnp.float32)
    ones1 = jnp.ones((CHUNK, 1), jnp.float32)

    deg_parts = _deg_kernel(dst2, zeros1, ones1)
    hp1, dis = _tc1(x, W1, deg_parts)
    parts1 = _agg_kernel(hp1, src2, dst2, zerosD)
    hp2 = _tc2(parts1, hp1, dis, b1.reshape(1, D), W2)
    parts2 = _agg_kernel(hp2, src2, dst2, zerosD)
    out = _tc3(parts2, hp2, dis, b2.reshape(1, D))
    return out


# speedup vs baseline: 20.8352x; 20.8352x over previous
import functools

import jax
import jax.numpy as jnp
from jax import lax
from jax.experimental import pallas as pl
from jax.experimental.pallas import tpu as pltpu
from jax.experimental.pallas import tpu_sc as plsc

N = 10000
E = 320000
D = 128

NC = 2
NS = 16
NW = NC * NS
CHUNK = 125
CPW = E // (NW * CHUNK)
RPT = 624
TAIL_BASE = NS * RPT
TAIL = N - TAIL_BASE


def _tile_rows_copy(s, mk_src, mk_dst):
    pltpu.sync_copy(mk_src(s * RPT, RPT), mk_dst(s * RPT, RPT))

    @pl.when(s == NS - 1)
    def _():
        pltpu.sync_copy(mk_src(TAIL_BASE, TAIL), mk_dst(TAIL_BASE, TAIL))

_SC_MESH = plsc.VectorSubcoreMesh(core_axis_name="c", subcore_axis_name="s")


def _deg_body(dst_hbm, zeros_hbm, ones_hbm, out_hbm, idx_v, ones_v, deg_sh,
              sem):
    c = lax.axis_index("c")
    s = lax.axis_index("s")
    wid = c * NS + s
    pltpu.sync_copy(ones_hbm, ones_v)
    _tile_rows_copy(s, lambda b, n: zeros_hbm.at[pl.ds(b, n)],
                    lambda b, n: deg_sh.at[pl.ds(b, n)])
    cp = pltpu.async_copy(dst_hbm.at[pl.ds(wid * CPW, CPW)], idx_v, sem)
    cp.wait()
    plsc.subcore_barrier()

    def body(j, carry):
        pltpu.sync_copy(ones_v, deg_sh.at[idx_v.at[j]], add=True)
        return carry

    lax.fori_loop(0, CPW, body, 0)
    plsc.subcore_barrier()
    _tile_rows_copy(s, lambda b, n: deg_sh.at[pl.ds(b, n)],
                    lambda b, n: out_hbm.at[c, pl.ds(b, n)])


_deg_kernel = pl.kernel(
    _deg_body,
    out_type=jax.ShapeDtypeStruct((NC, N, 1), jnp.float32),
    mesh=_SC_MESH,
    scratch_types=[
        pltpu.VMEM((CPW, CHUNK), jnp.int32),
        pltpu.VMEM((CHUNK, 1), jnp.float32),
        pltpu.VMEM_SHARED((N, 1), jnp.float32),
        pltpu.SemaphoreType.DMA,
    ],
)


def _agg_body(h_hbm, src_hbm, dst_hbm, zeros_hbm, out_hbm,
              isrc_v, idst_v, buf0, acc_sh, sem0, isem):
    c = lax.axis_index("c")
    s = lax.axis_index("s")
    wid = c * NS + s
    cp_s = pltpu.async_copy(src_hbm.at[pl.ds(wid * CPW, CPW)], isrc_v, isem)
    _tile_rows_copy(s, lambda b, n: zeros_hbm.at[pl.ds(b, n)],
                    lambda b, n: acc_sh.at[pl.ds(b, n)])
    pltpu.sync_copy(dst_hbm.at[pl.ds(wid * CPW, CPW)], idst_v)
    cp_s.wait()
    plsc.subcore_barrier()

    def body(j, carry):
        pltpu.async_copy(h_hbm.at[isrc_v.at[j]], buf0, sem0).wait()
        pltpu.sync_copy(buf0, acc_sh.at[idst_v.at[j]], add=True)
        return carry

    lax.fori_loop(0, CPW, body, 0)
    plsc.subcore_barrier()
    _tile_rows_copy(s, lambda b, n: acc_sh.at[pl.ds(b, n)],
                    lambda b, n: out_hbm.at[c, pl.ds(b, n)])


_agg_kernel = pl.kernel(
    _agg_body,
    out_type=jax.ShapeDtypeStruct((NC, N, D), jnp.float32),
    mesh=_SC_MESH,
    scratch_types=[
        pltpu.VMEM((CPW, CHUNK), jnp.int32),
        pltpu.VMEM((CPW, CHUNK), jnp.int32),
        pltpu.VMEM((CHUNK, D), jnp.float32),
        pltpu.VMEM_SHARED((N, D), jnp.float32),
        pltpu.SemaphoreType.DMA,
        pltpu.SemaphoreType.DMA,
    ],
)


RB = 400
GRID = N // RB


def _tc1_body(x_ref, w_ref, dp_ref, hp_ref, dis_ref):
    dis = lax.rsqrt(dp_ref[0] + dp_ref[1] + 1.0)
    h = jnp.dot(x_ref[...], w_ref[...], preferred_element_type=jnp.float32)
    hp_ref[...] = h * dis
    dis_ref[...] = dis


_tc1 = pl.pallas_call(
    _tc1_body,
    grid=(GRID,),
    in_specs=[
        pl.BlockSpec((RB, D), lambda i: (i, 0)),
        pl.BlockSpec((D, D), lambda i: (0, 0)),
        pl.BlockSpec((NC, RB, 1), lambda i: (0, i, 0)),
    ],
    out_specs=[
        pl.BlockSpec((RB, D), lambda i: (i, 0)),
        pl.BlockSpec((RB, 1), lambda i: (i, 0)),
    ],
    out_shape=[
        jax.ShapeDtypeStruct((N, D), jnp.float32),
        jax.ShapeDtypeStruct((N, 1), jnp.float32),
    ],
)


def _tc2_body(p_ref, hp1_ref, dis_ref, b1_ref, w2_ref, hp2_ref):
    ssum = p_ref[0] + p_ref[1] + hp1_ref[...]
    h1 = jnp.maximum(ssum * dis_ref[...] + b1_ref[...], 0.0)
    h2 = jnp.dot(h1, w2_ref[...], preferred_element_type=jnp.float32)
    hp2_ref[...] = h2 * dis_ref[...]


_tc2 = pl.pallas_call(
    _tc2_body,
    grid=(GRID,),
    in_specs=[
        pl.BlockSpec((NC, RB, D), lambda i: (0, i, 0)),
        pl.BlockSpec((RB, D), lambda i: (i, 0)),
        pl.BlockSpec((RB, 1), lambda i: (i, 0)),
        pl.BlockSpec((1, D), lambda i: (0, 0)),
        pl.BlockSpec((D, D), lambda i: (0, 0)),
    ],
    out_specs=pl.BlockSpec((RB, D), lambda i: (i, 0)),
    out_shape=jax.ShapeDtypeStruct((N, D), jnp.float32),
)


def _tc3_body(p_ref, hp2_ref, dis_ref, b2_ref, out_ref):
    ssum = p_ref[0] + p_ref[1] + hp2_ref[...]
    out_ref[...] = ssum * dis_ref[...] + b2_ref[...]


_tc3 = pl.pallas_call(
    _tc3_body,
    grid=(GRID,),
    in_specs=[
        pl.BlockSpec((NC, RB, D), lambda i: (0, i, 0)),
        pl.BlockSpec((RB, D), lambda i: (i, 0)),
        pl.BlockSpec((RB, 1), lambda i: (i, 0)),
        pl.BlockSpec((1, D), lambda i: (0, 0)),
    ],
    out_specs=pl.BlockSpec((RB, D), lambda i: (i, 0)),
    out_shape=jax.ShapeDtypeStruct((N, D), jnp.float32),
)


@jax.jit
def kernel(x, edge_index, W1, b1, W2, b2):
    src2 = edge_index[0].reshape(E // CHUNK, CHUNK)
    dst2 = edge_index[1].reshape(E // CHUNK, CHUNK)
    zeros1 = jnp.zeros((N, 1), jnp.float32)
    zerosD = jnp.zeros((N, D), jnp.float32)
    ones1 = jnp.ones((CHUNK, 1), jnp.float32)

    deg_parts = _deg_kernel(dst2, zeros1, ones1)
    hp1, dis = _tc1(x, W1, deg_parts)
    parts1 = _agg_kernel(hp1, src2, dst2, zerosD)
    hp2 = _tc2(parts1, hp1, dis, b1.reshape(1, D), W2)
    parts2 = _agg_kernel(hp2, src2, dst2, zerosD)
    out = _tc3(parts2, hp2, dis, b2.reshape(1, D))
    return out
